# baseline (speedup 1.0000x reference)
import functools

import jax
import jax.numpy as jnp
from jax import lax
from jax.experimental import pallas as pl
from jax.experimental.pallas import tpu as pltpu
from jax.experimental.pallas import tpu_sc as plsc

_VOCAB = 1000000
_EMB = 64
_B = 1024
_L = 200

_NC = 2
_NS = 16
_NW = _NC * _NS
_ROWS_PER_W = (_B * _L) // _NW
_SEQS_PER_W = _ROWS_PER_W // _L
_HALF = _L // 2


def _pe_table() -> jax.Array:
    seq_index = jnp.arange(_L, dtype=jnp.float32).reshape(-1, 1)
    even_index = jnp.arange(0, _EMB, 2)
    denominator = jnp.power(10000.0, even_index.astype(jnp.float32) / _EMB)
    args_sc = seq_index / denominator
    pe = jnp.zeros((_L, _EMB), dtype=jnp.float32)
    pe = pe.at[:, even_index].set(jnp.sin(args_sc))
    pe = pe.at[:, even_index + 1].set(jnp.cos(args_sc))
    return pe


def _body(table_hbm, idx_hbm, pe_hbm, out_hbm, idx_v, pe_v, rows_v, sem0, sem1):
    wid = lax.axis_index("s") * _NC + lax.axis_index("c")

    pltpu.sync_copy(idx_hbm.at[wid], idx_v)
    pltpu.sync_copy(pe_hbm, pe_v)

    @pl.loop(0, _SEQS_PER_W)
    def _seq(s):
        cp0 = pltpu.make_async_copy(
            table_hbm.at[idx_v.at[2 * s]], rows_v.at[pl.ds(0, _HALF)], sem0)
        cp1 = pltpu.make_async_copy(
            table_hbm.at[idx_v.at[2 * s + 1]], rows_v.at[pl.ds(_HALF, _HALF)], sem1)
        cp0.start()
        cp1.start()
        cp0.wait()
        cp1.wait()

        @pl.loop(0, _L)
        def _row(r):
            for k in range(_EMB // 16):
                sl = pl.ds(k * 16, 16)
                rows_v[r, sl] = rows_v[r, sl] + pe_v[r, sl]

        row0 = wid * _ROWS_PER_W + s * _L
        pltpu.sync_copy(rows_v, out_hbm.at[pl.ds(row0, _L)])


def kernel(X, table):
    idx = X.reshape(_NW, 2 * _SEQS_PER_W, _HALF)
    pe = _pe_table()
    mesh = plsc.VectorSubcoreMesh(core_axis_name="c", subcore_axis_name="s")
    out = pl.kernel(
        _body,
        out_type=jax.ShapeDtypeStruct((_B * _L, _EMB), jnp.float32),
        mesh=mesh,
        scratch_types=[
            pltpu.VMEM((2 * _SEQS_PER_W, _HALF), jnp.int32),
            pltpu.VMEM((_L, _EMB), jnp.float32),
            pltpu.VMEM((_L, _EMB), jnp.float32),
            pltpu.SemaphoreType.DMA,
            pltpu.SemaphoreType.DMA,
        ],
        compiler_params=pltpu.CompilerParams(use_tc_tiling_on_sc=False),
    )(table, idx, pe)
    return out.reshape(_B, _L, _EMB)


# speedup vs baseline: 1.0556x; 1.0556x over previous
import jax
import jax.numpy as jnp
from jax import lax
from jax.experimental import pallas as pl
from jax.experimental.pallas import tpu as pltpu
from jax.experimental.pallas import tpu_sc as plsc

_EMB = 64
_B = 1024
_L = 200

_NC = 2
_NS = 16
_NW = _NC * _NS
_ROWS_PER_W = (_B * _L) // _NW
_CHUNK = _L // 2
_NCHUNK = _ROWS_PER_W // _CHUNK
_NSLOT = 12
_PREFETCH = 8


def _pe_table() -> jax.Array:
    seq_index = jnp.arange(_L, dtype=jnp.float32).reshape(-1, 1)
    even_index = jnp.arange(0, _EMB, 2)
    denominator = jnp.power(10000.0, even_index.astype(jnp.float32) / _EMB)
    args_sc = seq_index / denominator
    pe = jnp.zeros((_L, _EMB), dtype=jnp.float32)
    pe = pe.at[:, even_index].set(jnp.sin(args_sc))
    pe = pe.at[:, even_index + 1].set(jnp.cos(args_sc))
    return pe


def _body(table_hbm, idx_hbm, pe_hbm, out_hbm, idx_v, pe_v, rows_v, *sems):
    gsem = sems[:_NSLOT]
    osem = sems[_NSLOT:]
    wid = lax.axis_index("s") * _NC + lax.axis_index("c")

    pltpu.sync_copy(idx_hbm.at[wid], idx_v)
    pltpu.sync_copy(pe_hbm, pe_v)

    def start_gather(c):
        s = c % _NSLOT
        pltpu.make_async_copy(
            table_hbm.at[idx_v.at[c]], rows_v.at[s], gsem[s]).start()

    def wait_gather(c):
        s = c % _NSLOT
        pltpu.make_async_copy(
            table_hbm.at[idx_v.at[c]], rows_v.at[s], gsem[s]).wait()

    def out_copy(c):
        s = c % _NSLOT
        row0 = wid * _ROWS_PER_W + c * _CHUNK
        return pltpu.make_async_copy(
            rows_v.at[s], out_hbm.at[pl.ds(row0, _CHUNK)], osem[s])

    for c in range(_PREFETCH):
        start_gather(c)

    for c in range(_NCHUNK):
        s = c % _NSLOT
        wait_gather(c)

        pbase = (c % 2) * _CHUNK

        @pl.loop(0, _CHUNK)
        def _row(r, s=s, pbase=pbase):
            for k in range(_EMB // 16):
                sl = pl.ds(k * 16, 16)
                rows_v[s, r, sl] = rows_v[s, r, sl] + pe_v[pbase + r, sl]

        out_copy(c).start()

        nc = c + _PREFETCH
        if nc < _NCHUNK:
            if nc >= _NSLOT:
                out_copy(nc - _NSLOT).wait()
            start_gather(nc)

    for c in range(_NCHUNK - _NSLOT, _NCHUNK):
        out_copy(c).wait()


def kernel(X, table):
    idx = X.reshape(_NW, _NCHUNK, _CHUNK)
    pe = _pe_table()
    mesh = plsc.VectorSubcoreMesh(core_axis_name="c", subcore_axis_name="s")
    out = pl.kernel(
        _body,
        out_type=jax.ShapeDtypeStruct((_B * _L, _EMB), jnp.float32),
        mesh=mesh,
        scratch_types=[
            pltpu.VMEM((_NCHUNK, _CHUNK), jnp.int32),
            pltpu.VMEM((_L, _EMB), jnp.float32),
            pltpu.VMEM((_NSLOT, _CHUNK, _EMB), jnp.float32),
        ] + [pltpu.SemaphoreType.DMA] * (2 * _NSLOT),
        compiler_params=pltpu.CompilerParams(use_tc_tiling_on_sc=False),
    )(table, idx, pe)
    return out.reshape(_B, _L, _EMB)
